# baseline (speedup 1.0000x reference)
import functools

import jax
import jax.numpy as jnp
from jax import lax
from jax.experimental import pallas as pl
from jax.experimental.pallas import tpu as pltpu
from jax.experimental.pallas import tpu_sc as plsc

N = 3072
D = 1024
NA = N // 3
RB = 768
AB = RB // 3
GRID = N // RB


@functools.lru_cache(maxsize=None)
def _make_sc_gather():
    info = plsc.get_sparse_core_info()
    nw = info.num_cores * info.num_subcores
    rw = N // nw
    aw = NA // nw
    DW = D // 2

    mesh = plsc.VectorSubcoreMesh(core_axis_name="c", subcore_axis_name="s")

    @functools.partial(
        pl.kernel,
        mesh=mesh,
        out_type=(
            jax.ShapeDtypeStruct((N, DW), jnp.int32),
            jax.ShapeDtypeStruct((NA, DW), jnp.int32),
        ),
        scratch_types=[
            pltpu.VMEM((rw,), jnp.int32),
            pltpu.VMEM((aw,), jnp.int32),
            pltpu.VMEM((rw, DW), jnp.int32),
            pltpu.VMEM((aw, DW), jnp.int32),
            pltpu.SemaphoreType.DMA,
            pltpu.SemaphoreType.DMA,
        ],
    )
    def gather_k(table_hbm, idx_hbm, aidx_hbm, out_hbm, aout_hbm,
                 idx_v, aidx_v, rows_v, arows_v, sem, asem):
        wid = lax.axis_index("s") * info.num_cores + lax.axis_index("c")
        base = wid * rw
        abase = wid * aw
        pltpu.sync_copy(idx_hbm.at[pl.ds(base, rw)], idx_v)
        pltpu.sync_copy(aidx_hbm.at[pl.ds(abase, aw)], aidx_v)
        cp = pltpu.async_copy(table_hbm.at[idx_v], rows_v, sem)
        acp = pltpu.async_copy(table_hbm.at[aidx_v], arows_v, asem)
        cp.wait()
        pltpu.sync_copy(rows_v, out_hbm.at[pl.ds(base, rw)])
        acp.wait()
        pltpu.sync_copy(arows_v, aout_hbm.at[pl.ds(abase, aw)])

    return gather_k


def _normalize_body(e_ref, out_ref):
    e = e_ref[...]
    nrm = jnp.sqrt(jnp.sum(e * e, axis=1, keepdims=True))
    out_ref[...] = (e / jnp.maximum(nrm, 1e-12)).astype(jnp.bfloat16)


def _dist_body(a_ref, b_ref, an_ref, dist_ref, ranks_ref, medr_ref):
    i = pl.program_id(0)
    r0 = i * RB
    a = a_ref[...]
    b = b_ref[...]
    d = 1.0 - lax.dot_general(
        a, b, (((1,), (1,)), ((), ())), preferred_element_type=jnp.float32
    )
    dist_ref[...] = d

    lr = lax.broadcasted_iota(jnp.int32, (RB, RB), 0)
    lc = lax.broadcasted_iota(jnp.int32, (RB, RB), 1)
    strip = dist_ref[:, pl.ds(r0, RB)]
    dist_ref[:, pl.ds(r0, RB)] = jnp.where(
        ((lr % 3) == 0) & (lc == lr), -1.0, strip)

    an = an_ref[...]
    da = 1.0 - lax.dot_general(
        an, b, (((1,), (1,)), ((), ())), preferred_element_type=jnp.float32
    )
    col = lax.broadcasted_iota(jnp.int32, (AB, N), 1)
    acol = r0 + 3 * lax.broadcasted_iota(jnp.int32, (AB, 1), 0)
    tcol = acol + 1
    da = jnp.where(col == acol, -1.0, da)
    v = jnp.sum(jnp.where(col == tcol, da, 0.0), axis=1, keepdims=True)
    less = jnp.sum((da < v).astype(jnp.float32), axis=1, keepdims=True)
    eqb = jnp.sum(((da == v) & (col < tcol)).astype(jnp.float32),
                  axis=1, keepdims=True)
    ranks = less + eqb - 1.0
    ranks_ref[...] = ranks.astype(jnp.int32)

    part = jnp.sum(ranks)

    @pl.when(i == 0)
    def _():
        medr_ref[0, 0] = part

    @pl.when(i > 0)
    def _():
        medr_ref[0, 0] = medr_ref[0, 0] + part

    @pl.when(i == GRID - 1)
    def _():
        medr_ref[0, 0] = medr_ref[0, 0] / float(NA)


def kernel(embeddings, originalIndexes):
    normed = pl.pallas_call(
        _normalize_body,
        grid=(GRID,),
        in_specs=[pl.BlockSpec((RB, D), lambda i: (i, 0))],
        out_specs=pl.BlockSpec((RB, D), lambda i: (i, 0)),
        out_shape=jax.ShapeDtypeStruct((N, D), jnp.bfloat16),
    )(embeddings)

    aidx = originalIndexes[0::3]
    normed_w = lax.bitcast_convert_type(
        normed.reshape(N, D // 2, 2), jnp.int32)
    eW, aW = _make_sc_gather()(normed_w, originalIndexes, aidx)
    eN = lax.bitcast_convert_type(eW, jnp.bfloat16).reshape(N, D)
    aN = lax.bitcast_convert_type(aW, jnp.bfloat16).reshape(NA, D)

    dist, ranks2d, medr = pl.pallas_call(
        _dist_body,
        grid=(GRID,),
        in_specs=[
            pl.BlockSpec((RB, D), lambda i: (i, 0)),
            pl.BlockSpec((N, D), lambda i: (0, 0)),
            pl.BlockSpec((AB, D), lambda i: (i, 0)),
        ],
        out_specs=[
            pl.BlockSpec((RB, N), lambda i: (i, 0)),
            pl.BlockSpec((AB, 1), lambda i: (i, 0)),
            pl.BlockSpec(memory_space=pltpu.SMEM),
        ],
        out_shape=[
            jax.ShapeDtypeStruct((N, N), jnp.float32),
            jax.ShapeDtypeStruct((NA, 1), jnp.int32),
            jax.ShapeDtypeStruct((1, 1), jnp.float32),
        ],
    )(eN, eN, aN)

    return dist, ranks2d.reshape(NA), medr[0, 0]


# speedup vs baseline: 2.9413x; 2.9413x over previous
import functools

import jax
import jax.numpy as jnp
from jax import lax
from jax.experimental import pallas as pl
from jax.experimental.pallas import tpu as pltpu
from jax.experimental.pallas import tpu_sc as plsc

N = 3072
D = 1024
NA = N // 3
RB = 768
AB = RB // 3
GRID = N // RB


@functools.lru_cache(maxsize=None)
def _make_sc_gather():
    info = plsc.get_sparse_core_info()
    nw = info.num_cores * info.num_subcores
    rw = N // nw
    aw = NA // nw

    mesh = plsc.VectorSubcoreMesh(core_axis_name="c", subcore_axis_name="s")

    @functools.partial(
        pl.kernel,
        mesh=mesh,
        out_type=(
            jax.ShapeDtypeStruct((N, D), jnp.float32),
            jax.ShapeDtypeStruct((NA, D), jnp.float32),
        ),
        scratch_types=[
            pltpu.VMEM((rw,), jnp.int32),
            pltpu.VMEM((aw,), jnp.int32),
            pltpu.VMEM((rw, D), jnp.float32),
            pltpu.SemaphoreType.DMA,
        ],
    )
    def gather_k(table_hbm, idx_hbm, aidx_hbm, out_hbm, aout_hbm,
                 idx_v, aidx_v, rows_v, sem):
        wid = lax.axis_index("s") * info.num_cores + lax.axis_index("c")
        base = wid * rw
        abase = wid * aw
        pltpu.sync_copy(idx_hbm.at[pl.ds(base, rw)], idx_v)
        pltpu.sync_copy(aidx_hbm.at[pl.ds(abase, aw)], aidx_v)
        pltpu.async_copy(table_hbm.at[idx_v], rows_v, sem).wait()
        pltpu.sync_copy(rows_v, out_hbm.at[pl.ds(base, rw)])
        arows = rows_v.at[pl.ds(0, aw)]
        pltpu.async_copy(table_hbm.at[aidx_v], arows, sem).wait()
        pltpu.sync_copy(arows, aout_hbm.at[pl.ds(abase, aw)])

    return gather_k


def _normalize_body(e_ref, out_ref):
    e = e_ref[...]
    nrm = jnp.sqrt(jnp.sum(e * e, axis=1, keepdims=True))
    out_ref[...] = (e / jnp.maximum(nrm, 1e-12)).astype(jnp.bfloat16)


def _dist_body(a_ref, b_ref, an_ref, dist_ref, ranks_ref, medr_ref):
    i = pl.program_id(0)
    r0 = i * RB
    a = a_ref[...]
    b = b_ref[...]
    d = 1.0 - lax.dot_general(
        a, b, (((1,), (1,)), ((), ())), preferred_element_type=jnp.float32
    )
    dist_ref[...] = d

    lr = lax.broadcasted_iota(jnp.int32, (RB, RB), 0)
    lc = lax.broadcasted_iota(jnp.int32, (RB, RB), 1)
    strip = dist_ref[:, pl.ds(r0, RB)]
    dist_ref[:, pl.ds(r0, RB)] = jnp.where(
        ((lr % 3) == 0) & (lc == lr), -1.0, strip)

    an = an_ref[...]
    da = 1.0 - lax.dot_general(
        an, b, (((1,), (1,)), ((), ())), preferred_element_type=jnp.float32
    )
    col = lax.broadcasted_iota(jnp.int32, (AB, N), 1)
    acol = r0 + 3 * lax.broadcasted_iota(jnp.int32, (AB, 1), 0)
    tcol = acol + 1
    da = jnp.where(col == acol, -1.0, da)
    v = jnp.sum(jnp.where(col == tcol, da, 0.0), axis=1, keepdims=True)
    less = jnp.sum((da < v).astype(jnp.float32), axis=1, keepdims=True)
    eqb = jnp.sum(((da == v) & (col < tcol)).astype(jnp.float32),
                  axis=1, keepdims=True)
    ranks = less + eqb - 1.0
    ranks_ref[...] = ranks.astype(jnp.int32)

    part = jnp.sum(ranks)

    @pl.when(i == 0)
    def _():
        medr_ref[0, 0] = part

    @pl.when(i > 0)
    def _():
        medr_ref[0, 0] = medr_ref[0, 0] + part

    @pl.when(i == GRID - 1)
    def _():
        medr_ref[0, 0] = medr_ref[0, 0] / float(NA)


def kernel(embeddings, originalIndexes):
    aidx = originalIndexes[0::3]
    e, ea = _make_sc_gather()(embeddings, originalIndexes, aidx)

    eN = pl.pallas_call(
        _normalize_body,
        grid=(GRID,),
        in_specs=[pl.BlockSpec((RB, D), lambda i: (i, 0))],
        out_specs=pl.BlockSpec((RB, D), lambda i: (i, 0)),
        out_shape=jax.ShapeDtypeStruct((N, D), jnp.bfloat16),
    )(e)

    aN = pl.pallas_call(
        _normalize_body,
        grid=(2,),
        in_specs=[pl.BlockSpec((NA // 2, D), lambda i: (i, 0))],
        out_specs=pl.BlockSpec((NA // 2, D), lambda i: (i, 0)),
        out_shape=jax.ShapeDtypeStruct((NA, D), jnp.bfloat16),
    )(ea)

    dist, ranks2d, medr = pl.pallas_call(
        _dist_body,
        grid=(GRID,),
        in_specs=[
            pl.BlockSpec((RB, D), lambda i: (i, 0)),
            pl.BlockSpec((N, D), lambda i: (0, 0)),
            pl.BlockSpec((AB, D), lambda i: (i, 0)),
        ],
        out_specs=[
            pl.BlockSpec((RB, N), lambda i: (i, 0)),
            pl.BlockSpec((AB, 1), lambda i: (i, 0)),
            pl.BlockSpec(memory_space=pltpu.SMEM),
        ],
        out_shape=[
            jax.ShapeDtypeStruct((N, N), jnp.float32),
            jax.ShapeDtypeStruct((NA, 1), jnp.int32),
            jax.ShapeDtypeStruct((1, 1), jnp.float32),
        ],
    )(eN, eN, aN)

    return dist, ranks2d.reshape(NA), medr[0, 0]


# speedup vs baseline: 3.1730x; 1.0788x over previous
import functools

import jax
import jax.numpy as jnp
from jax import lax
from jax.experimental import pallas as pl
from jax.experimental.pallas import tpu as pltpu
from jax.experimental.pallas import tpu_sc as plsc

N = 3072
D = 1024
NA = N // 3
RB = 768
AB = RB // 3
PH = N // RB


@functools.lru_cache(maxsize=None)
def _make_sc_gather():
    info = plsc.get_sparse_core_info()
    nw = info.num_cores * info.num_subcores
    rw = N // nw
    aw = NA // nw

    mesh = plsc.VectorSubcoreMesh(core_axis_name="c", subcore_axis_name="s")

    @functools.partial(
        pl.kernel,
        mesh=mesh,
        out_type=(
            jax.ShapeDtypeStruct((N, D), jnp.float32),
            jax.ShapeDtypeStruct((NA, D), jnp.float32),
        ),
        scratch_types=[
            pltpu.VMEM((rw,), jnp.int32),
            pltpu.VMEM((aw,), jnp.int32),
            pltpu.VMEM((rw, D), jnp.float32),
            pltpu.SemaphoreType.DMA,
        ],
    )
    def gather_k(table_hbm, idx_hbm, aidx_hbm, out_hbm, aout_hbm,
                 idx_v, aidx_v, rows_v, sem):
        wid = lax.axis_index("s") * info.num_cores + lax.axis_index("c")
        base = wid * rw
        abase = wid * aw
        pltpu.sync_copy(idx_hbm.at[pl.ds(base, rw)], idx_v)
        pltpu.sync_copy(aidx_hbm.at[pl.ds(abase, aw)], aidx_v)
        pltpu.async_copy(table_hbm.at[idx_v], rows_v, sem).wait()
        pltpu.sync_copy(rows_v, out_hbm.at[pl.ds(base, rw)])
        arows = rows_v.at[pl.ds(0, aw)]
        pltpu.async_copy(table_hbm.at[aidx_v], arows, sem).wait()
        pltpu.sync_copy(arows, aout_hbm.at[pl.ds(abase, aw)])

    return gather_k


def _normed_bf16(e):
    nrm = jnp.sqrt(jnp.sum(e * e, axis=1, keepdims=True))
    return (e / jnp.maximum(nrm, 1e-12)).astype(jnp.bfloat16)


def _fused_body(e_ref, ea_ref, dist_ref, ranks_ref, medr_ref, nb_ref, na_ref):
    i = pl.program_id(0)

    @pl.when(i < PH)
    def _normalize_phase():
        nb_ref[pl.ds(i * RB, RB), :] = _normed_bf16(e_ref[...])
        na_ref[pl.ds(i * AB, AB), :] = _normed_bf16(ea_ref[...])

    @pl.when(i >= PH)
    def _dist_phase():
        j = i - PH
        r0 = j * RB
        a = nb_ref[pl.ds(r0, RB), :]
        b = nb_ref[...]
        d = 1.0 - lax.dot_general(
            a, b, (((1,), (1,)), ((), ())),
            preferred_element_type=jnp.float32)
        dist_ref[...] = d

        lr = lax.broadcasted_iota(jnp.int32, (RB, RB), 0)
        lc = lax.broadcasted_iota(jnp.int32, (RB, RB), 1)
        strip = dist_ref[:, pl.ds(r0, RB)]
        dist_ref[:, pl.ds(r0, RB)] = jnp.where(
            ((lr % 3) == 0) & (lc == lr), -1.0, strip)

        an = na_ref[pl.ds(j * AB, AB), :]
        da = 1.0 - lax.dot_general(
            an, b, (((1,), (1,)), ((), ())),
            preferred_element_type=jnp.float32)
        col = lax.broadcasted_iota(jnp.int32, (AB, N), 1)
        acol = r0 + 3 * lax.broadcasted_iota(jnp.int32, (AB, 1), 0)
        tcol = acol + 1
        da = jnp.where(col == acol, -1.0, da)
        v = jnp.sum(jnp.where(col == tcol, da, 0.0), axis=1, keepdims=True)
        less = jnp.sum((da < v).astype(jnp.float32), axis=1, keepdims=True)
        eqb = jnp.sum(((da == v) & (col < tcol)).astype(jnp.float32),
                      axis=1, keepdims=True)
        ranks = less + eqb - 1.0
        ranks_ref[...] = ranks.astype(jnp.int32)

        part = jnp.sum(ranks)

        @pl.when(j == 0)
        def _():
            medr_ref[0, 0] = part

        @pl.when(j > 0)
        def _():
            medr_ref[0, 0] = medr_ref[0, 0] + part

        @pl.when(j == PH - 1)
        def _():
            medr_ref[0, 0] = medr_ref[0, 0] / float(NA)


def kernel(embeddings, originalIndexes):
    aidx = originalIndexes[0::3]
    e, ea = _make_sc_gather()(embeddings, originalIndexes, aidx)

    dist, ranks2d, medr = pl.pallas_call(
        _fused_body,
        grid=(2 * PH,),
        in_specs=[
            pl.BlockSpec((RB, D), lambda i: (jnp.minimum(i, PH - 1), 0)),
            pl.BlockSpec((AB, D), lambda i: (jnp.minimum(i, PH - 1), 0)),
        ],
        out_specs=[
            pl.BlockSpec((RB, N), lambda i: (jnp.maximum(i - PH, 0), 0)),
            pl.BlockSpec((AB, 1), lambda i: (jnp.maximum(i - PH, 0), 0)),
            pl.BlockSpec(memory_space=pltpu.SMEM),
        ],
        out_shape=[
            jax.ShapeDtypeStruct((N, N), jnp.float32),
            jax.ShapeDtypeStruct((NA, 1), jnp.int32),
            jax.ShapeDtypeStruct((1, 1), jnp.float32),
        ],
        scratch_shapes=[
            pltpu.VMEM((N, D), jnp.bfloat16),
            pltpu.VMEM((NA, D), jnp.bfloat16),
        ],
    )(e, ea)

    return dist, ranks2d.reshape(NA), medr[0, 0]


# speedup vs baseline: 3.3744x; 1.0635x over previous
import functools

import jax
import jax.numpy as jnp
from jax import lax
from jax.experimental import pallas as pl
from jax.experimental.pallas import tpu as pltpu
from jax.experimental.pallas import tpu_sc as plsc

N = 3072
D = 1024
NA = N // 3
RB = 768
AB = RB // 3
PH = N // RB


@functools.lru_cache(maxsize=None)
def _make_sc_gather():
    info = plsc.get_sparse_core_info()
    nw = info.num_cores * info.num_subcores
    rw = N // nw
    aw = NA // nw

    mesh = plsc.VectorSubcoreMesh(core_axis_name="c", subcore_axis_name="s")

    hw = rw // 2

    @functools.partial(
        pl.kernel,
        mesh=mesh,
        out_type=jax.ShapeDtypeStruct((N, D), jnp.float32),
        scratch_types=[
            pltpu.VMEM((rw,), jnp.int32),
            pltpu.VMEM((rw, D), jnp.float32),
            pltpu.SemaphoreType.DMA,
            pltpu.SemaphoreType.DMA,
            pltpu.SemaphoreType.DMA,
            pltpu.SemaphoreType.DMA,
        ],
    )
    def gather_k(table_hbm, idx_hbm, out_hbm,
                 idx_v, rows_v, sg0, sg1, ss0, ss1):
        wid = lax.axis_index("s") * info.num_cores + lax.axis_index("c")
        base = wid * rw
        pltpu.sync_copy(idx_hbm.at[pl.ds(base, rw)], idx_v)
        g0 = pltpu.async_copy(
            table_hbm.at[idx_v.at[pl.ds(0, hw)]], rows_v.at[pl.ds(0, hw)], sg0)
        g1 = pltpu.async_copy(
            table_hbm.at[idx_v.at[pl.ds(hw, hw)]], rows_v.at[pl.ds(hw, hw)], sg1)
        g0.wait()
        s0 = pltpu.async_copy(
            rows_v.at[pl.ds(0, hw)], out_hbm.at[pl.ds(base, hw)], ss0)
        g1.wait()
        s1 = pltpu.async_copy(
            rows_v.at[pl.ds(hw, hw)], out_hbm.at[pl.ds(base + hw, hw)], ss1)
        s0.wait()
        s1.wait()

    return gather_k


def _normed_bf16(e):
    nrm = jnp.sqrt(jnp.sum(e * e, axis=1, keepdims=True))
    return (e / jnp.maximum(nrm, 1e-12)).astype(jnp.bfloat16)


def _fused_body(e_ref, dist_ref, ranks_ref, medr_ref, nb_ref, na_ref):
    i = pl.program_id(0)

    @pl.when(i < PH)
    def _normalize_phase():
        nb = _normed_bf16(e_ref[...])
        nb_ref[pl.ds(i * RB, RB), :] = nb
        ja = lax.broadcasted_iota(jnp.int32, (AB, RB), 0)
        ka = lax.broadcasted_iota(jnp.int32, (AB, RB), 1)
        sel = (ka == 3 * ja).astype(jnp.bfloat16)
        na_ref[pl.ds(i * AB, AB), :] = lax.dot_general(
            sel, nb, (((1,), (0,)), ((), ())),
            preferred_element_type=jnp.float32).astype(jnp.bfloat16)

    @pl.when(i >= PH)
    def _dist_phase():
        j = i - PH
        r0 = j * RB
        a = nb_ref[pl.ds(r0, RB), :]
        b = nb_ref[...]
        d = 1.0 - lax.dot_general(
            a, b, (((1,), (1,)), ((), ())),
            preferred_element_type=jnp.float32)
        dist_ref[...] = d

        lr = lax.broadcasted_iota(jnp.int32, (RB, RB), 0)
        lc = lax.broadcasted_iota(jnp.int32, (RB, RB), 1)
        strip = dist_ref[:, pl.ds(r0, RB)]
        dist_ref[:, pl.ds(r0, RB)] = jnp.where(
            ((lr % 3) == 0) & (lc == lr), -1.0, strip)

        an = na_ref[pl.ds(j * AB, AB), :]
        da = 1.0 - lax.dot_general(
            an, b, (((1,), (1,)), ((), ())),
            preferred_element_type=jnp.float32)
        col = lax.broadcasted_iota(jnp.int32, (AB, N), 1)
        acol = r0 + 3 * lax.broadcasted_iota(jnp.int32, (AB, 1), 0)
        tcol = acol + 1
        da = jnp.where(col == acol, -1.0, da)
        v = jnp.sum(jnp.where(col == tcol, da, 0.0), axis=1, keepdims=True)
        less = jnp.sum((da < v).astype(jnp.float32), axis=1, keepdims=True)
        eqb = jnp.sum(((da == v) & (col < tcol)).astype(jnp.float32),
                      axis=1, keepdims=True)
        ranks = less + eqb - 1.0
        ranks_ref[...] = ranks.astype(jnp.int32)

        part = jnp.sum(ranks)

        @pl.when(j == 0)
        def _():
            medr_ref[0, 0] = part

        @pl.when(j > 0)
        def _():
            medr_ref[0, 0] = medr_ref[0, 0] + part

        @pl.when(j == PH - 1)
        def _():
            medr_ref[0, 0] = medr_ref[0, 0] / float(NA)


def kernel(embeddings, originalIndexes):
    e = _make_sc_gather()(embeddings, originalIndexes)

    dist, ranks2d, medr = pl.pallas_call(
        _fused_body,
        grid=(2 * PH,),
        in_specs=[
            pl.BlockSpec((RB, D), lambda i: (jnp.minimum(i, PH - 1), 0)),
        ],
        out_specs=[
            pl.BlockSpec((RB, N), lambda i: (jnp.maximum(i - PH, 0), 0)),
            pl.BlockSpec((AB, 1), lambda i: (jnp.maximum(i - PH, 0), 0)),
            pl.BlockSpec(memory_space=pltpu.SMEM),
        ],
        out_shape=[
            jax.ShapeDtypeStruct((N, N), jnp.float32),
            jax.ShapeDtypeStruct((NA, 1), jnp.int32),
            jax.ShapeDtypeStruct((1, 1), jnp.float32),
        ],
        scratch_shapes=[
            pltpu.VMEM((N, D), jnp.bfloat16),
            pltpu.VMEM((NA, D), jnp.bfloat16),
        ],
    )(e)

    return dist, ranks2d.reshape(NA), medr[0, 0]
